# baseline (speedup 1.0000x reference)
import jax
import jax.numpy as jnp
from jax.experimental import pallas as pl
from jax.experimental.pallas import tpu as pltpu

_N = 2048
_IN = 16384
_K = 40
_BLK = 256
_NB = _N // _BLK


def _fused_body(inp_ref, conn_ref, out_ref, ov_blk, ov_col):
    s = pl.program_id(0)
    ov = jnp.sum(conn_ref[:] * inp_ref[:], axis=1)
    ov_blk[pl.ds(s, 1), :] = ov.reshape(1, _BLK)
    ov_col[pl.ds(s * _BLK, _BLK), :] = ov.reshape(_BLK, 1)

    @pl.when(s == _NB - 1)
    def _rank_and_mask():
        orow = ov_blk[:].reshape(1, _N)
        tri = (
            jax.lax.broadcasted_iota(jnp.int32, (_BLK, _BLK), 1)
            < jax.lax.broadcasted_iota(jnp.int32, (_BLK, _BLK), 0)
        )
        for b in range(_NB):
            lo, hi = b * _BLK, (b + 1) * _BLK
            oc = ov_col[lo:hi, :]
            rank = jnp.zeros((_BLK, 1), jnp.float32)
            if b > 0:
                rank = jnp.sum(
                    jnp.where(orow[:, :lo] >= oc, 1.0, 0.0),
                    axis=1, keepdims=True,
                )
            rank = rank + jnp.sum(
                jnp.where(orow[:, lo:] > oc, 1.0, 0.0),
                axis=1, keepdims=True,
            )
            rank = rank + jnp.sum(
                jnp.where((orow[:, lo:hi] == oc) & tri, 1.0, 0.0),
                axis=1, keepdims=True,
            )
            out_ref[lo:hi, :] = (rank < float(_K)).astype(jnp.float32)


def kernel(input_vector, connections):
    mask = pl.pallas_call(
        _fused_body,
        grid=(_NB,),
        in_specs=[
            pl.BlockSpec((1, _IN), lambda i: (0, 0)),
            pl.BlockSpec((_BLK, _IN), lambda i: (i, 0)),
        ],
        out_specs=pl.BlockSpec((_N, 1), lambda i: (0, 0)),
        out_shape=jax.ShapeDtypeStruct((_N, 1), jnp.float32),
        scratch_shapes=[
            pltpu.VMEM((_NB, _BLK), jnp.float32),
            pltpu.VMEM((_N, 1), jnp.float32),
        ],
    )(input_vector.reshape(1, _IN), connections)
    return mask.reshape(_N)
